# Initial kernel scaffold; baseline (speedup 1.0000x reference)
import jax
import jax.numpy as jnp
from jax.experimental import pallas as pl


def kernel(edge_index, features, edge_weights, W0, b0, W1, b1, W2, b2):
    raise NotImplementedError("write your pallas kernel here")


# speedup vs baseline: 26.4746x; 26.4746x over previous
import functools

import jax
import jax.numpy as jnp
from jax import lax
from jax.experimental import pallas as pl
from jax.experimental.pallas import tpu as pltpu
from jax.experimental.pallas import tpu_sc as plsc

NC = 2
NS = 16
NW = NC * NS
LANES = 16
CHUNK = 1024
CROWS = CHUNK // 128


@functools.lru_cache(maxsize=None)
def _make_agg(n_nodes, d, nchunks):
    rows_per_tile = n_nodes // NS
    mesh = plsc.VectorSubcoreMesh(
        core_axis_name="c", subcore_axis_name="s", num_cores=NC, num_subcores=NS
    )

    @functools.partial(
        pl.kernel,
        out_type=jax.ShapeDtypeStruct((NC, n_nodes, d), jnp.float32),
        mesh=mesh,
        scratch_types=[
            pltpu.VMEM_SHARED((n_nodes, d), jnp.float32),
            pltpu.VMEM((CROWS, 128), jnp.int32),
            pltpu.VMEM((CROWS, 128), jnp.int32),
            pltpu.VMEM((CHUNK,), jnp.float32),
            pltpu.VMEM((CHUNK, d), jnp.float32),
            pltpu.SemaphoreType.DMA,
        ],
        compiler_params=pltpu.CompilerParams(use_tc_tiling_on_sc=False),
    )
    def agg(zeros_hbm, xw_hbm, row_hbm, col_hbm, wts_hbm, out_hbm,
            acc, row_v, col_v, wts_v, msg_v, sem):
        cid = lax.axis_index("c")
        sid = lax.axis_index("s")
        wid = sid * NC + cid

        pltpu.sync_copy(
            zeros_hbm.at[pl.ds(sid * rows_per_tile, rows_per_tile)],
            acc.at[pl.ds(sid * rows_per_tile, rows_per_tile)],
        )
        plsc.subcore_barrier()

        def chunk_body(i, carry):
            crow = (wid * nchunks + i) * CROWS
            ebase = (wid * nchunks + i) * CHUNK
            pltpu.sync_copy(row_hbm.at[pl.ds(crow, CROWS)], row_v)
            pltpu.sync_copy(col_hbm.at[pl.ds(crow, CROWS)], col_v)
            pltpu.sync_copy(wts_hbm.at[pl.ds(ebase, CHUNK)], wts_v)

            cps = [
                pltpu.async_copy(
                    xw_hbm.at[row_v.at[j]],
                    msg_v.at[pl.ds(j * 128, 128)],
                    sem,
                )
                for j in range(CROWS)
            ]
            for cp in cps:
                cp.wait()

            def scale_body(k, c2):
                w16 = wts_v[pl.ds(k * LANES, LANES)]
                base = k * LANES
                for j in range(LANES):
                    wj = lax.broadcast_in_dim(w16[j], (LANES,), ())
                    msg_v[base + j, :] = msg_v[base + j, :] * wj
                return c2

            lax.fori_loop(0, CHUNK // LANES, scale_body, 0)

            for j in range(CROWS):
                pltpu.sync_copy(
                    msg_v.at[pl.ds(j * 128, 128)],
                    acc.at[col_v.at[j]],
                    add=True,
                )
            return carry

        lax.fori_loop(0, nchunks, chunk_body, 0)
        plsc.subcore_barrier()

        pltpu.sync_copy(
            acc.at[pl.ds(sid * rows_per_tile, rows_per_tile)],
            out_hbm.at[cid, pl.ds(sid * rows_per_tile, rows_per_tile)],
        )

    return agg


def _pick_blk(n):
    for b in range(2048, 7, -1):
        if n % b == 0 and b % 8 == 0:
            return b
    return n


def _mm_body(x_ref, w_ref, o_ref):
    o_ref[...] = jnp.dot(x_ref[...], w_ref[...], preferred_element_type=jnp.float32)


def _matmul(x, w):
    n, k = x.shape
    m = w.shape[1]
    blk = _pick_blk(n)
    return pl.pallas_call(
        _mm_body,
        grid=(n // blk,),
        in_specs=[
            pl.BlockSpec((blk, k), lambda i: (i, 0)),
            pl.BlockSpec((k, m), lambda i: (0, 0)),
        ],
        out_specs=pl.BlockSpec((blk, m), lambda i: (i, 0)),
        out_shape=jax.ShapeDtypeStruct((n, m), jnp.float32),
    )(x, w)


def _relu_mm_body(p_ref, b_ref, w_ref, o_ref):
    h = jnp.maximum(p_ref[0] + p_ref[1] + b_ref[...], 0.0)
    o_ref[...] = jnp.dot(h, w_ref[...], preferred_element_type=jnp.float32)


def _relu_matmul(p, b, w):
    _, n, d = p.shape
    m = w.shape[1]
    blk = _pick_blk(n)
    return pl.pallas_call(
        _relu_mm_body,
        grid=(n // blk,),
        in_specs=[
            pl.BlockSpec((2, blk, d), lambda i: (0, i, 0)),
            pl.BlockSpec((1, d), lambda i: (0, 0)),
            pl.BlockSpec((d, m), lambda i: (0, 0)),
        ],
        out_specs=pl.BlockSpec((blk, m), lambda i: (i, 0)),
        out_shape=jax.ShapeDtypeStruct((n, m), jnp.float32),
    )(p, b.reshape(1, d), w)


def _relu_body(p_ref, b_ref, o_ref):
    o_ref[...] = jnp.maximum(p_ref[0] + p_ref[1] + b_ref[...], 0.0)


def _relu_bias(p, b):
    _, n, d = p.shape
    blk = _pick_blk(n)
    return pl.pallas_call(
        _relu_body,
        grid=(n // blk,),
        in_specs=[
            pl.BlockSpec((2, blk, d), lambda i: (0, i, 0)),
            pl.BlockSpec((1, d), lambda i: (0, 0)),
        ],
        out_specs=pl.BlockSpec((blk, d), lambda i: (i, 0)),
        out_shape=jax.ShapeDtypeStruct((n, d), jnp.float32),
    )(p, b.reshape(1, d))


def _final_body(p_ref, w_ref, b_ref, o_ref):
    z = jnp.dot(p_ref[0] + p_ref[1], w_ref[...], preferred_element_type=jnp.float32)
    z = z + b_ref[...]
    m = jnp.max(z, axis=1, keepdims=True)
    e = jnp.exp(z - m)
    s = jnp.sum(e, axis=1, keepdims=True)
    o_ref[...] = z - m - jnp.log(s)


def _final(p, w, b):
    _, n, d = p.shape
    m = w.shape[1]
    blk = _pick_blk(n)
    return pl.pallas_call(
        _final_body,
        grid=(n // blk,),
        in_specs=[
            pl.BlockSpec((2, blk, d), lambda i: (0, i, 0)),
            pl.BlockSpec((d, m), lambda i: (0, 0)),
            pl.BlockSpec((1, m), lambda i: (0, 0)),
        ],
        out_specs=pl.BlockSpec((blk, m), lambda i: (i, 0)),
        out_shape=jax.ShapeDtypeStruct((n, m), jnp.float32),
    )(p, w, b.reshape(1, m))


def kernel(edge_index, features, edge_weights, W0, b0, W1, b1, W2, b2):
    n_nodes, _ = features.shape
    n_edges = edge_index.shape[1]
    d = W0.shape[1]

    n_nodes_p = -(-n_nodes // (NS * 8)) * (NS * 8)

    per_w = -(-n_edges // (NW * CHUNK)) * CHUNK
    nchunks = per_w // CHUNK
    n_pad = NW * per_w
    pad = n_pad - n_edges

    row = jnp.concatenate([edge_index[0], jnp.zeros((pad,), jnp.int32)])
    col = jnp.concatenate([edge_index[1], jnp.zeros((pad,), jnp.int32)])
    wts = jnp.concatenate([edge_weights, jnp.zeros((pad,), jnp.float32)])
    row2 = row.reshape(-1, 128)
    col2 = col.reshape(-1, 128)
    zeros = jnp.zeros((n_nodes_p, d), jnp.float32)

    agg = _make_agg(n_nodes_p, d, nchunks)

    xw0 = _matmul(features, W0)
    p0 = agg(zeros, xw0, row2, col2, wts)
    xw1 = _relu_matmul(p0, b0, W1)
    p1 = agg(zeros, xw1, row2, col2, wts)
    h1 = _relu_bias(p1, b1)
    p2 = agg(zeros, h1, row2, col2, wts)
    out = _final(p2, W2, b2)
    return out[:n_nodes]


# speedup vs baseline: 32.2584x; 1.2185x over previous
import functools

import jax
import jax.numpy as jnp
from jax import lax
from jax.experimental import pallas as pl
from jax.experimental.pallas import tpu as pltpu
from jax.experimental.pallas import tpu_sc as plsc

NC = 2
NS = 16
NW = NC * NS
LANES = 16
CHUNK = 768
CROWS = CHUNK // 128


@functools.lru_cache(maxsize=None)
def _make_agg(n_nodes, d, nchunks):
    rows_per_tile = n_nodes // NS
    mesh = plsc.VectorSubcoreMesh(
        core_axis_name="c", subcore_axis_name="s", num_cores=NC, num_subcores=NS
    )

    @functools.partial(
        pl.kernel,
        out_type=jax.ShapeDtypeStruct((NC, n_nodes, d), jnp.float32),
        mesh=mesh,
        scratch_types=[
            pltpu.VMEM_SHARED((n_nodes, d), jnp.float32),
            pltpu.VMEM((2, CROWS, 128), jnp.int32),
            pltpu.VMEM((2, CROWS, 128), jnp.int32),
            pltpu.VMEM((2, CHUNK), jnp.float32),
            pltpu.VMEM((2, CHUNK, d), jnp.float32),
            pltpu.SemaphoreType.DMA,
            pltpu.SemaphoreType.DMA,
            pltpu.SemaphoreType.DMA,
            pltpu.SemaphoreType.DMA,
        ],
        compiler_params=pltpu.CompilerParams(use_tc_tiling_on_sc=False),
    )
    def agg(zeros_hbm, xw_hbm, row_hbm, col_hbm, wts_hbm, out_hbm,
            acc, row_v, col_v, wts_v, msg_v, gsem0, gsem1, ssem0, ssem1):
        cid = lax.axis_index("c")
        sid = lax.axis_index("s")
        wid = sid * NC + cid
        gsem = (gsem0, gsem1)
        ssem = (ssem0, ssem1)

        pltpu.sync_copy(
            zeros_hbm.at[pl.ds(sid * rows_per_tile, rows_per_tile)],
            acc.at[pl.ds(sid * rows_per_tile, rows_per_tile)],
        )
        plsc.subcore_barrier()

        def stage_and_fire_gather(i, b):
            crow = (wid * nchunks + i) * CROWS
            ebase = (wid * nchunks + i) * CHUNK
            pltpu.sync_copy(row_hbm.at[pl.ds(crow, CROWS)], row_v.at[b])
            pltpu.sync_copy(col_hbm.at[pl.ds(crow, CROWS)], col_v.at[b])
            pltpu.sync_copy(wts_hbm.at[pl.ds(ebase, CHUNK)], wts_v.at[b])
            for j in range(CROWS):
                pltpu.async_copy(
                    xw_hbm.at[row_v.at[b, j]],
                    msg_v.at[b, pl.ds(j * 128, 128)],
                    gsem[b],
                )

        def wait_gather(b):
            for j in range(CROWS):
                pltpu.make_async_copy(
                    xw_hbm.at[row_v.at[b, j]],
                    msg_v.at[b, pl.ds(j * 128, 128)],
                    gsem[b],
                ).wait()

        def fire_scatter(b):
            for j in range(CROWS):
                pltpu.async_copy(
                    msg_v.at[b, pl.ds(j * 128, 128)],
                    acc.at[col_v.at[b, j]],
                    ssem[b],
                    add=True,
                )

        def drain_scatter(b):
            for j in range(CROWS):
                pltpu.make_async_copy(
                    msg_v.at[b, pl.ds(j * 128, 128)],
                    acc.at[col_v.at[b, j]],
                    ssem[b],
                ).wait()

        def scale(b):
            def scale_body(k, c2):
                w16 = wts_v[b, pl.ds(k * LANES, LANES)]
                base = k * LANES
                for j in range(LANES):
                    wj = lax.broadcast_in_dim(w16[j], (LANES,), ())
                    msg_v[b, base + j, :] = msg_v[b, base + j, :] * wj
                return c2

            lax.fori_loop(0, CHUNK // LANES, scale_body, 0)

        def step(a, b):
            ob = 1 - b

            @pl.when(a > 0)
            def _():
                drain_scatter(ob)

            @pl.when(a + 1 < nchunks)
            def _():
                stage_and_fire_gather(a + 1, ob)

            wait_gather(b)
            scale(b)
            fire_scatter(b)

        stage_and_fire_gather(0, 0)

        def pair_body(t, carry):
            step(2 * t, 0)

            @pl.when(2 * t + 1 < nchunks)
            def _():
                step(2 * t + 1, 1)

            return carry

        lax.fori_loop(0, (nchunks + 1) // 2, pair_body, 0)
        drain_scatter((nchunks - 1) % 2)
        plsc.subcore_barrier()

        pltpu.sync_copy(
            acc.at[pl.ds(sid * rows_per_tile, rows_per_tile)],
            out_hbm.at[cid, pl.ds(sid * rows_per_tile, rows_per_tile)],
        )

    return agg


def _pick_blk(n):
    for b in range(2048, 7, -1):
        if n % b == 0 and b % 8 == 0:
            return b
    return n


def _mm_body(x_ref, w_ref, o_ref):
    o_ref[...] = jnp.dot(x_ref[...], w_ref[...], preferred_element_type=jnp.float32)


def _matmul(x, w):
    n, k = x.shape
    m = w.shape[1]
    blk = _pick_blk(n)
    return pl.pallas_call(
        _mm_body,
        grid=(n // blk,),
        in_specs=[
            pl.BlockSpec((blk, k), lambda i: (i, 0)),
            pl.BlockSpec((k, m), lambda i: (0, 0)),
        ],
        out_specs=pl.BlockSpec((blk, m), lambda i: (i, 0)),
        out_shape=jax.ShapeDtypeStruct((n, m), jnp.float32),
    )(x, w)


def _relu_mm_body(p_ref, b_ref, w_ref, o_ref):
    h = jnp.maximum(p_ref[0] + p_ref[1] + b_ref[...], 0.0)
    o_ref[...] = jnp.dot(h, w_ref[...], preferred_element_type=jnp.float32)


def _relu_matmul(p, b, w):
    _, n, d = p.shape
    m = w.shape[1]
    blk = _pick_blk(n)
    return pl.pallas_call(
        _relu_mm_body,
        grid=(n // blk,),
        in_specs=[
            pl.BlockSpec((2, blk, d), lambda i: (0, i, 0)),
            pl.BlockSpec((1, d), lambda i: (0, 0)),
            pl.BlockSpec((d, m), lambda i: (0, 0)),
        ],
        out_specs=pl.BlockSpec((blk, m), lambda i: (i, 0)),
        out_shape=jax.ShapeDtypeStruct((n, m), jnp.float32),
    )(p, b.reshape(1, d), w)


def _relu_body(p_ref, b_ref, o_ref):
    o_ref[...] = jnp.maximum(p_ref[0] + p_ref[1] + b_ref[...], 0.0)


def _relu_bias(p, b):
    _, n, d = p.shape
    blk = _pick_blk(n)
    return pl.pallas_call(
        _relu_body,
        grid=(n // blk,),
        in_specs=[
            pl.BlockSpec((2, blk, d), lambda i: (0, i, 0)),
            pl.BlockSpec((1, d), lambda i: (0, 0)),
        ],
        out_specs=pl.BlockSpec((blk, d), lambda i: (i, 0)),
        out_shape=jax.ShapeDtypeStruct((n, d), jnp.float32),
    )(p, b.reshape(1, d))


def _final_body(p_ref, w_ref, b_ref, o_ref):
    z = jnp.dot(p_ref[0] + p_ref[1], w_ref[...], preferred_element_type=jnp.float32)
    z = z + b_ref[...]
    m = jnp.max(z, axis=1, keepdims=True)
    e = jnp.exp(z - m)
    s = jnp.sum(e, axis=1, keepdims=True)
    o_ref[...] = z - m - jnp.log(s)


def _final(p, w, b):
    _, n, d = p.shape
    m = w.shape[1]
    blk = _pick_blk(n)
    return pl.pallas_call(
        _final_body,
        grid=(n // blk,),
        in_specs=[
            pl.BlockSpec((2, blk, d), lambda i: (0, i, 0)),
            pl.BlockSpec((d, m), lambda i: (0, 0)),
            pl.BlockSpec((1, m), lambda i: (0, 0)),
        ],
        out_specs=pl.BlockSpec((blk, m), lambda i: (i, 0)),
        out_shape=jax.ShapeDtypeStruct((n, m), jnp.float32),
    )(p, w, b.reshape(1, m))


def kernel(edge_index, features, edge_weights, W0, b0, W1, b1, W2, b2):
    n_nodes, _ = features.shape
    n_edges = edge_index.shape[1]
    d = W0.shape[1]

    n_nodes_p = -(-n_nodes // (NS * 8)) * (NS * 8)

    per_w = -(-n_edges // (NW * CHUNK)) * CHUNK
    nchunks = per_w // CHUNK
    n_pad = NW * per_w
    pad = n_pad - n_edges

    row = jnp.concatenate([edge_index[0], jnp.zeros((pad,), jnp.int32)])
    col = jnp.concatenate([edge_index[1], jnp.zeros((pad,), jnp.int32)])
    wts = jnp.concatenate([edge_weights, jnp.zeros((pad,), jnp.float32)])
    row2 = row.reshape(-1, 128)
    col2 = col.reshape(-1, 128)
    zeros = jnp.zeros((n_nodes_p, d), jnp.float32)

    agg = _make_agg(n_nodes_p, d, nchunks)

    xw0 = _matmul(features, W0)
    p0 = agg(zeros, xw0, row2, col2, wts)
    xw1 = _relu_matmul(p0, b0, W1)
    p1 = agg(zeros, xw1, row2, col2, wts)
    h1 = _relu_bias(p1, b1)
    p2 = agg(zeros, h1, row2, col2, wts)
    out = _final(p2, W2, b2)
    return out[:n_nodes]


# speedup vs baseline: 35.5728x; 1.1027x over previous
import functools

import jax
import jax.numpy as jnp
from jax import lax
from jax.experimental import pallas as pl
from jax.experimental.pallas import tpu as pltpu
from jax.experimental.pallas import tpu_sc as plsc

NC = 2
NS = 16
NW = NC * NS
LANES = 16
CHUNK = 768
CROWS = CHUNK // 128


@functools.lru_cache(maxsize=None)
def _make_agg(n_nodes, d, nchunks):
    rows_per_tile = n_nodes // NS
    mesh = plsc.VectorSubcoreMesh(
        core_axis_name="c", subcore_axis_name="s", num_cores=NC, num_subcores=NS
    )

    @functools.partial(
        pl.kernel,
        out_type=jax.ShapeDtypeStruct((NC, n_nodes, d), jnp.float32),
        mesh=mesh,
        scratch_types=[
            pltpu.VMEM_SHARED((n_nodes, d), jnp.float32),
            pltpu.VMEM((2, 3 * CROWS, 128), jnp.int32),
            pltpu.VMEM((2, CHUNK, d), jnp.float32),
            pltpu.SemaphoreType.DMA,
            pltpu.SemaphoreType.DMA,
            pltpu.SemaphoreType.DMA,
            pltpu.SemaphoreType.DMA,
        ],
        compiler_params=pltpu.CompilerParams(
            use_tc_tiling_on_sc=False, needs_layout_passes=False
        ),
    )
    def agg(zeros_hbm, xw_hbm, pk_hbm, out_hbm,
            acc, stg_v, msg_v, gsem0, gsem1, ssem0, ssem1):
        cid = lax.axis_index("c")
        sid = lax.axis_index("s")
        wid = sid * NC + cid
        gsem = (gsem0, gsem1)
        ssem = (ssem0, ssem1)

        pltpu.sync_copy(
            zeros_hbm.at[pl.ds(sid * rows_per_tile, rows_per_tile)],
            acc.at[pl.ds(sid * rows_per_tile, rows_per_tile)],
        )
        plsc.subcore_barrier()

        def stage_and_fire_gather(i, b):
            ci = wid * nchunks + i
            pltpu.sync_copy(pk_hbm.at[ci], stg_v.at[b])
            for j in range(CROWS):
                pltpu.async_copy(
                    xw_hbm.at[stg_v.at[b, j]],
                    msg_v.at[b, pl.ds(j * 128, 128)],
                    gsem[b],
                )

        def wait_gather(b):
            for j in range(CROWS):
                pltpu.make_async_copy(
                    xw_hbm.at[stg_v.at[b, j]],
                    msg_v.at[b, pl.ds(j * 128, 128)],
                    gsem[b],
                ).wait()

        def fire_scatter(b):
            for j in range(CROWS):
                pltpu.async_copy(
                    msg_v.at[b, pl.ds(j * 128, 128)],
                    acc.at[stg_v.at[b, CROWS + j]],
                    ssem[b],
                    add=True,
                )

        def drain_scatter(b):
            for j in range(CROWS):
                pltpu.make_async_copy(
                    msg_v.at[b, pl.ds(j * 128, 128)],
                    acc.at[stg_v.at[b, CROWS + j]],
                    ssem[b],
                ).wait()

        def scale(b):
            def scale_body(k, c2):
                w16i = stg_v[b, 2 * CROWS + k // 8, pl.ds((k % 8) * LANES, LANES)]
                w16 = plsc.bitcast(w16i, jnp.float32)
                base = k * LANES
                for j in range(LANES):
                    wj = lax.gather(
                        w16,
                        jnp.full((LANES, 1), j, jnp.int32),
                        lax.GatherDimensionNumbers(
                            offset_dims=(),
                            collapsed_slice_dims=(0,),
                            start_index_map=(0,),
                        ),
                        (1,),
                        mode=lax.GatherScatterMode.PROMISE_IN_BOUNDS,
                    )
                    msg_v[b, base + j, :] = msg_v[b, base + j, :] * wj
                return c2

            lax.fori_loop(0, CHUNK // LANES, scale_body, 0)

        def step(a, b):
            ob = 1 - b

            @pl.when(a > 0)
            def _():
                drain_scatter(ob)

            @pl.when(a + 1 < nchunks)
            def _():
                stage_and_fire_gather(a + 1, ob)

            wait_gather(b)
            scale(b)
            fire_scatter(b)

        stage_and_fire_gather(0, 0)

        def pair_body(t, carry):
            step(2 * t, 0)

            @pl.when(2 * t + 1 < nchunks)
            def _():
                step(2 * t + 1, 1)

            return carry

        lax.fori_loop(0, (nchunks + 1) // 2, pair_body, 0)
        drain_scatter((nchunks - 1) % 2)
        plsc.subcore_barrier()

        pltpu.sync_copy(
            acc.at[pl.ds(sid * rows_per_tile, rows_per_tile)],
            out_hbm.at[cid, pl.ds(sid * rows_per_tile, rows_per_tile)],
        )

    return agg


def _pick_blk(n):
    for b in range(2048, 7, -1):
        if n % b == 0 and b % 8 == 0:
            return b
    return n


def _mm_body(x_ref, w_ref, o_ref):
    o_ref[...] = jnp.dot(x_ref[...], w_ref[...], preferred_element_type=jnp.float32)


def _matmul(x, w):
    n, k = x.shape
    m = w.shape[1]
    blk = _pick_blk(n)
    return pl.pallas_call(
        _mm_body,
        grid=(n // blk,),
        in_specs=[
            pl.BlockSpec((blk, k), lambda i: (i, 0)),
            pl.BlockSpec((k, m), lambda i: (0, 0)),
        ],
        out_specs=pl.BlockSpec((blk, m), lambda i: (i, 0)),
        out_shape=jax.ShapeDtypeStruct((n, m), jnp.float32),
    )(x, w)


def _relu_mm_body(p_ref, b_ref, w_ref, o_ref):
    h = jnp.maximum(p_ref[0] + p_ref[1] + b_ref[...], 0.0)
    o_ref[...] = jnp.dot(h, w_ref[...], preferred_element_type=jnp.float32)


def _relu_matmul(p, b, w):
    _, n, d = p.shape
    m = w.shape[1]
    blk = _pick_blk(n)
    return pl.pallas_call(
        _relu_mm_body,
        grid=(n // blk,),
        in_specs=[
            pl.BlockSpec((2, blk, d), lambda i: (0, i, 0)),
            pl.BlockSpec((1, d), lambda i: (0, 0)),
            pl.BlockSpec((d, m), lambda i: (0, 0)),
        ],
        out_specs=pl.BlockSpec((blk, m), lambda i: (i, 0)),
        out_shape=jax.ShapeDtypeStruct((n, m), jnp.float32),
    )(p, b.reshape(1, d), w)


def _relu_body(p_ref, b_ref, o_ref):
    o_ref[...] = jnp.maximum(p_ref[0] + p_ref[1] + b_ref[...], 0.0)


def _relu_bias(p, b):
    _, n, d = p.shape
    blk = _pick_blk(n)
    return pl.pallas_call(
        _relu_body,
        grid=(n // blk,),
        in_specs=[
            pl.BlockSpec((2, blk, d), lambda i: (0, i, 0)),
            pl.BlockSpec((1, d), lambda i: (0, 0)),
        ],
        out_specs=pl.BlockSpec((blk, d), lambda i: (i, 0)),
        out_shape=jax.ShapeDtypeStruct((n, d), jnp.float32),
    )(p, b.reshape(1, d))


def _final_body(p_ref, w_ref, b_ref, o_ref):
    z = jnp.dot(p_ref[0] + p_ref[1], w_ref[...], preferred_element_type=jnp.float32)
    z = z + b_ref[...]
    m = jnp.max(z, axis=1, keepdims=True)
    e = jnp.exp(z - m)
    s = jnp.sum(e, axis=1, keepdims=True)
    o_ref[...] = z - m - jnp.log(s)


def _final(p, w, b):
    _, n, d = p.shape
    m = w.shape[1]
    blk = _pick_blk(n)
    return pl.pallas_call(
        _final_body,
        grid=(n // blk,),
        in_specs=[
            pl.BlockSpec((2, blk, d), lambda i: (0, i, 0)),
            pl.BlockSpec((d, m), lambda i: (0, 0)),
            pl.BlockSpec((1, m), lambda i: (0, 0)),
        ],
        out_specs=pl.BlockSpec((blk, m), lambda i: (i, 0)),
        out_shape=jax.ShapeDtypeStruct((n, m), jnp.float32),
    )(p, w, b.reshape(1, m))


def kernel(edge_index, features, edge_weights, W0, b0, W1, b1, W2, b2):
    n_nodes, _ = features.shape
    n_edges = edge_index.shape[1]
    d = W0.shape[1]

    n_nodes_p = -(-n_nodes // (NS * 8)) * (NS * 8)

    per_w = -(-n_edges // (NW * CHUNK)) * CHUNK
    nchunks = per_w // CHUNK
    n_pad = NW * per_w
    pad = n_pad - n_edges

    row = jnp.concatenate([edge_index[0], jnp.zeros((pad,), jnp.int32)])
    col = jnp.concatenate([edge_index[1], jnp.zeros((pad,), jnp.int32)])
    wts = jnp.concatenate([edge_weights, jnp.zeros((pad,), jnp.float32)])
    ntot = n_pad // CHUNK
    row3 = row.reshape(ntot, CROWS, 128)
    col3 = col.reshape(ntot, CROWS, 128)
    wts3 = lax.bitcast_convert_type(wts, jnp.int32).reshape(ntot, CROWS, 128)
    packed = jnp.concatenate([row3, col3, wts3], axis=1)
    zeros = jnp.zeros((n_nodes_p, d), jnp.float32)

    agg = _make_agg(n_nodes_p, d, nchunks)

    xw0 = _matmul(features, W0)
    p0 = agg(zeros, xw0, packed)
    xw1 = _relu_matmul(p0, b0, W1)
    p1 = agg(zeros, xw1, packed)
    h1 = _relu_bias(p1, b1)
    p2 = agg(zeros, h1, packed)
    out = _final(p2, W2, b2)
    return out[:n_nodes]


# speedup vs baseline: 36.5910x; 1.0286x over previous
import functools

import jax
import jax.numpy as jnp
from jax import lax
from jax.experimental import pallas as pl
from jax.experimental.pallas import tpu as pltpu
from jax.experimental.pallas import tpu_sc as plsc

NC = 2
NS = 16
NW = NC * NS
LANES = 16
CHUNK = 768
CROWS = CHUNK // 128
PKROWS = -(-3 * CROWS // 8) * 8


@functools.lru_cache(maxsize=None)
def _make_agg(n_nodes, d, nchunks):
    rows_per_tile = n_nodes // NS
    mesh = plsc.VectorSubcoreMesh(
        core_axis_name="c", subcore_axis_name="s", num_cores=NC, num_subcores=NS
    )

    @functools.partial(
        pl.kernel,
        out_type=jax.ShapeDtypeStruct((NC, n_nodes, d), jnp.float32),
        mesh=mesh,
        scratch_types=[
            pltpu.VMEM_SHARED((n_nodes, d), jnp.float32),
            pltpu.VMEM((2, PKROWS, 128), jnp.int32),
            pltpu.VMEM((2, CHUNK, d), jnp.float32),
            pltpu.SemaphoreType.DMA,
            pltpu.SemaphoreType.DMA,
            pltpu.SemaphoreType.DMA,
            pltpu.SemaphoreType.DMA,
        ],
        compiler_params=pltpu.CompilerParams(
            use_tc_tiling_on_sc=False, needs_layout_passes=False
        ),
    )
    def agg(zeros_hbm, xw_hbm, pk_hbm, out_hbm,
            acc, stg_v, msg_v, gsem0, gsem1, ssem0, ssem1):
        cid = lax.axis_index("c")
        sid = lax.axis_index("s")
        wid = sid * NC + cid
        gsem = (gsem0, gsem1)
        ssem = (ssem0, ssem1)

        pltpu.sync_copy(
            zeros_hbm.at[pl.ds(sid * rows_per_tile, rows_per_tile)],
            acc.at[pl.ds(sid * rows_per_tile, rows_per_tile)],
        )
        plsc.subcore_barrier()

        def stage_and_fire_gather(i, b):
            ci = wid * nchunks + i
            pltpu.sync_copy(pk_hbm.at[ci], stg_v.at[b])
            for j in range(CROWS):
                pltpu.async_copy(
                    xw_hbm.at[stg_v.at[b, j]],
                    msg_v.at[b, pl.ds(j * 128, 128)],
                    gsem[b],
                )

        def wait_gather(b):
            for j in range(CROWS):
                pltpu.make_async_copy(
                    xw_hbm.at[stg_v.at[b, j]],
                    msg_v.at[b, pl.ds(j * 128, 128)],
                    gsem[b],
                ).wait()

        def fire_scatter(b):
            for j in range(CROWS):
                pltpu.async_copy(
                    msg_v.at[b, pl.ds(j * 128, 128)],
                    acc.at[stg_v.at[b, CROWS + j]],
                    ssem[b],
                    add=True,
                )

        def drain_scatter(b):
            for j in range(CROWS):
                pltpu.make_async_copy(
                    msg_v.at[b, pl.ds(j * 128, 128)],
                    acc.at[stg_v.at[b, CROWS + j]],
                    ssem[b],
                ).wait()

        def scale(b):
            def scale_body(k, c2):
                w16i = stg_v[b, 2 * CROWS + k // 8, pl.ds((k % 8) * LANES, LANES)]
                w16 = plsc.bitcast(w16i, jnp.float32)
                base = k * LANES
                for j in range(LANES):
                    wj = lax.gather(
                        w16,
                        jnp.full((LANES, 1), j, jnp.int32),
                        lax.GatherDimensionNumbers(
                            offset_dims=(),
                            collapsed_slice_dims=(0,),
                            start_index_map=(0,),
                        ),
                        (1,),
                        mode=lax.GatherScatterMode.PROMISE_IN_BOUNDS,
                    )
                    msg_v[b, base + j, :] = msg_v[b, base + j, :] * wj
                return c2

            lax.fori_loop(0, CHUNK // LANES, scale_body, 0)

        def step(a, b):
            ob = 1 - b

            @pl.when(a > 0)
            def _():
                drain_scatter(ob)

            @pl.when(a + 1 < nchunks)
            def _():
                stage_and_fire_gather(a + 1, ob)

            wait_gather(b)
            scale(b)
            fire_scatter(b)

        stage_and_fire_gather(0, 0)

        def pair_body(t, carry):
            step(2 * t, 0)

            @pl.when(2 * t + 1 < nchunks)
            def _():
                step(2 * t + 1, 1)

            return carry

        lax.fori_loop(0, (nchunks + 1) // 2, pair_body, 0)
        drain_scatter((nchunks - 1) % 2)
        plsc.subcore_barrier()

        pltpu.sync_copy(
            acc.at[pl.ds(sid * rows_per_tile, rows_per_tile)],
            out_hbm.at[cid, pl.ds(sid * rows_per_tile, rows_per_tile)],
        )

    return agg


def _pick_blk(n):
    for b in range(2048, 7, -1):
        if n % b == 0 and b % 8 == 0:
            return b
    return n


def _mm_body(x_ref, w_ref, o_ref):
    o_ref[...] = jnp.dot(x_ref[...], w_ref[...], preferred_element_type=jnp.float32)


def _matmul(x, w):
    n, k = x.shape
    m = w.shape[1]
    blk = _pick_blk(n)
    return pl.pallas_call(
        _mm_body,
        grid=(n // blk,),
        in_specs=[
            pl.BlockSpec((blk, k), lambda i: (i, 0)),
            pl.BlockSpec((k, m), lambda i: (0, 0)),
        ],
        out_specs=pl.BlockSpec((blk, m), lambda i: (i, 0)),
        out_shape=jax.ShapeDtypeStruct((n, m), jnp.float32),
    )(x, w)


def _relu_mm_body(p_ref, b_ref, w_ref, o_ref):
    h = jnp.maximum(p_ref[0] + p_ref[1] + b_ref[...], 0.0)
    o_ref[...] = jnp.dot(h, w_ref[...], preferred_element_type=jnp.float32)


def _relu_matmul(p, b, w):
    _, n, d = p.shape
    m = w.shape[1]
    blk = _pick_blk(n)
    return pl.pallas_call(
        _relu_mm_body,
        grid=(n // blk,),
        in_specs=[
            pl.BlockSpec((2, blk, d), lambda i: (0, i, 0)),
            pl.BlockSpec((1, d), lambda i: (0, 0)),
            pl.BlockSpec((d, m), lambda i: (0, 0)),
        ],
        out_specs=pl.BlockSpec((blk, m), lambda i: (i, 0)),
        out_shape=jax.ShapeDtypeStruct((n, m), jnp.float32),
    )(p, b.reshape(1, d), w)


def _relu_body(p_ref, b_ref, o_ref):
    o_ref[...] = jnp.maximum(p_ref[0] + p_ref[1] + b_ref[...], 0.0)


def _relu_bias(p, b):
    _, n, d = p.shape
    blk = _pick_blk(n)
    return pl.pallas_call(
        _relu_body,
        grid=(n // blk,),
        in_specs=[
            pl.BlockSpec((2, blk, d), lambda i: (0, i, 0)),
            pl.BlockSpec((1, d), lambda i: (0, 0)),
        ],
        out_specs=pl.BlockSpec((blk, d), lambda i: (i, 0)),
        out_shape=jax.ShapeDtypeStruct((n, d), jnp.float32),
    )(p, b.reshape(1, d))


def _final_body(p_ref, w_ref, b_ref, o_ref):
    z = jnp.dot(p_ref[0] + p_ref[1], w_ref[...], preferred_element_type=jnp.float32)
    z = z + b_ref[...]
    m = jnp.max(z, axis=1, keepdims=True)
    e = jnp.exp(z - m)
    s = jnp.sum(e, axis=1, keepdims=True)
    o_ref[...] = z - m - jnp.log(s)


def _final(p, w, b):
    _, n, d = p.shape
    m = w.shape[1]
    blk = _pick_blk(n)
    return pl.pallas_call(
        _final_body,
        grid=(n // blk,),
        in_specs=[
            pl.BlockSpec((2, blk, d), lambda i: (0, i, 0)),
            pl.BlockSpec((d, m), lambda i: (0, 0)),
            pl.BlockSpec((1, m), lambda i: (0, 0)),
        ],
        out_specs=pl.BlockSpec((blk, m), lambda i: (i, 0)),
        out_shape=jax.ShapeDtypeStruct((n, m), jnp.float32),
    )(p, w, b.reshape(1, m))


def kernel(edge_index, features, edge_weights, W0, b0, W1, b1, W2, b2):
    n_nodes, _ = features.shape
    n_edges = edge_index.shape[1]
    d = W0.shape[1]

    n_nodes_p = -(-n_nodes // (NS * 8)) * (NS * 8)

    per_w = -(-n_edges // (NW * CHUNK)) * CHUNK
    nchunks = per_w // CHUNK
    n_pad = NW * per_w
    pad = n_pad - n_edges

    row = jnp.concatenate([edge_index[0], jnp.zeros((pad,), jnp.int32)])
    col = jnp.concatenate([edge_index[1], jnp.zeros((pad,), jnp.int32)])
    wts = jnp.concatenate([edge_weights, jnp.zeros((pad,), jnp.float32)])
    ntot = n_pad // CHUNK
    row3 = row.reshape(ntot, CROWS, 128)
    col3 = col.reshape(ntot, CROWS, 128)
    wts3 = lax.bitcast_convert_type(wts, jnp.int32).reshape(ntot, CROWS, 128)
    zpad = jnp.zeros((ntot, PKROWS - 3 * CROWS, 128), jnp.int32)
    packed = jnp.concatenate([row3, col3, wts3, zpad], axis=1)
    zeros = jnp.zeros((n_nodes_p, d), jnp.float32)

    agg = _make_agg(n_nodes_p, d, nchunks)

    xw0 = _matmul(features, W0)
    p0 = agg(zeros, xw0, packed)
    xw1 = _relu_matmul(p0, b0, W1)
    p1 = agg(zeros, xw1, packed)
    h1 = _relu_bias(p1, b1)
    p2 = agg(zeros, h1, packed)
    out = _final(p2, W2, b2)
    return out[:n_nodes]


# speedup vs baseline: 42.8936x; 1.1722x over previous
import functools

import jax
import jax.numpy as jnp
from jax import lax
from jax.experimental import pallas as pl
from jax.experimental.pallas import tpu as pltpu
from jax.experimental.pallas import tpu_sc as plsc

NC = 2
NS = 16
NW = NC * NS
LANES = 16
CHUNK = 768
CROWS = CHUNK // 128
PKROWS = -(-3 * CROWS // 8) * 8


@functools.lru_cache(maxsize=None)
def _make_agg(n_nodes, d, nchunks):
    rows_per_tile = n_nodes // NS
    mesh = plsc.VectorSubcoreMesh(
        core_axis_name="c", subcore_axis_name="s", num_cores=NC, num_subcores=NS
    )

    @functools.partial(
        pl.kernel,
        out_type=jax.ShapeDtypeStruct((NC, n_nodes, d), jnp.float32),
        mesh=mesh,
        scratch_types=[
            pltpu.VMEM_SHARED((n_nodes, d), jnp.float32),
            pltpu.VMEM((2, PKROWS, 128), jnp.int32),
            pltpu.VMEM((2, CHUNK, d), jnp.float32),
            pltpu.SemaphoreType.DMA,
            pltpu.SemaphoreType.DMA,
            pltpu.SemaphoreType.DMA,
            pltpu.SemaphoreType.DMA,
        ],
        compiler_params=pltpu.CompilerParams(
            use_tc_tiling_on_sc=False, needs_layout_passes=False
        ),
    )
    def agg(zeros_hbm, xw_hbm, pk_hbm, out_hbm,
            acc, stg_v, msg_v, gsem0, gsem1, ssem0, ssem1):
        cid = lax.axis_index("c")
        sid = lax.axis_index("s")
        wid = sid * NC + cid
        gsem = (gsem0, gsem1)
        ssem = (ssem0, ssem1)

        pltpu.sync_copy(
            zeros_hbm.at[pl.ds(sid * rows_per_tile, rows_per_tile)],
            acc.at[pl.ds(sid * rows_per_tile, rows_per_tile)],
        )
        plsc.subcore_barrier()

        def stage_and_fire_gather(i, b):
            ci = wid * nchunks + i
            pltpu.sync_copy(pk_hbm.at[ci], stg_v.at[b])
            for j in range(CROWS):
                pltpu.async_copy(
                    xw_hbm.at[stg_v.at[b, j]],
                    msg_v.at[b, pl.ds(j * 128, 128)],
                    gsem[b],
                )

        def wait_gather(b):
            for j in range(CROWS):
                pltpu.make_async_copy(
                    xw_hbm.at[stg_v.at[b, j]],
                    msg_v.at[b, pl.ds(j * 128, 128)],
                    gsem[b],
                ).wait()

        def fire_scatter(b):
            for j in range(CROWS):
                pltpu.async_copy(
                    msg_v.at[b, pl.ds(j * 128, 128)],
                    acc.at[stg_v.at[b, CROWS + j]],
                    ssem[b],
                    add=True,
                )

        def drain_scatter(b):
            for j in range(CROWS):
                pltpu.make_async_copy(
                    msg_v.at[b, pl.ds(j * 128, 128)],
                    acc.at[stg_v.at[b, CROWS + j]],
                    ssem[b],
                ).wait()

        def scale(b):
            def scale_body(k, c2):
                w16i = stg_v[b, 2 * CROWS + k // 8, pl.ds((k % 8) * LANES, LANES)]
                w16 = plsc.bitcast(w16i, jnp.float32)
                base = k * LANES
                for j in range(LANES):
                    wj = lax.gather(
                        w16,
                        jnp.full((LANES, 1), j, jnp.int32),
                        lax.GatherDimensionNumbers(
                            offset_dims=(),
                            collapsed_slice_dims=(0,),
                            start_index_map=(0,),
                        ),
                        (1,),
                        mode=lax.GatherScatterMode.PROMISE_IN_BOUNDS,
                    )
                    msg_v[b, base + j, :] = msg_v[b, base + j, :] * wj
                return c2

            lax.fori_loop(0, CHUNK // LANES, scale_body, 0)

        def step(a, b):
            ob = 1 - b

            @pl.when(a > 0)
            def _():
                drain_scatter(ob)

            @pl.when(a + 1 < nchunks)
            def _():
                stage_and_fire_gather(a + 1, ob)

            wait_gather(b)
            scale(b)
            fire_scatter(b)

        stage_and_fire_gather(0, 0)

        def pair_body(t, carry):
            step(2 * t, 0)

            @pl.when(2 * t + 1 < nchunks)
            def _():
                step(2 * t + 1, 1)

            return carry

        lax.fori_loop(0, (nchunks + 1) // 2, pair_body, 0)
        drain_scatter((nchunks - 1) % 2)
        plsc.subcore_barrier()

        pltpu.sync_copy(
            acc.at[pl.ds(sid * rows_per_tile, rows_per_tile)],
            out_hbm.at[cid, pl.ds(sid * rows_per_tile, rows_per_tile)],
        )

    return agg


def _pick_blk(n):
    for b in range(2048, 7, -1):
        if n % b == 0 and b % 8 == 0:
            return b
    return n


def _mm_body(x_ref, w_ref, o_ref):
    o_ref[...] = jnp.dot(x_ref[...], w_ref[...], preferred_element_type=jnp.float32)


def _matmul(x, w):
    n, k = x.shape
    m = w.shape[1]
    blk = _pick_blk(n)
    return pl.pallas_call(
        _mm_body,
        grid=(n // blk,),
        in_specs=[
            pl.BlockSpec((blk, k), lambda i: (i, 0)),
            pl.BlockSpec((k, m), lambda i: (0, 0)),
        ],
        out_specs=pl.BlockSpec((blk, m), lambda i: (i, 0)),
        out_shape=jax.ShapeDtypeStruct((n, m), jnp.float32),
    )(x, w)


def _relu_mm_body(p_ref, b_ref, w_ref, o_ref):
    h = jnp.maximum(p_ref[0] + p_ref[1] + b_ref[...], 0.0)
    o_ref[...] = jnp.dot(h, w_ref[...], preferred_element_type=jnp.float32)


def _relu_matmul_packed(p, b, wbd):
    _, n8, d8 = p.shape
    m8 = wbd.shape[1]
    blk = _pick_blk(n8)
    return pl.pallas_call(
        _relu_mm_body,
        grid=(n8 // blk,),
        in_specs=[
            pl.BlockSpec((2, blk, d8), lambda i: (0, i, 0)),
            pl.BlockSpec((1, d8), lambda i: (0, 0)),
            pl.BlockSpec((d8, m8), lambda i: (0, 0)),
        ],
        out_specs=pl.BlockSpec((blk, m8), lambda i: (i, 0)),
        out_shape=jax.ShapeDtypeStruct((n8, m8), jnp.float32),
    )(p, b.reshape(1, d8), wbd)


def _relu_body(p_ref, b_ref, o_ref):
    o_ref[...] = jnp.maximum(p_ref[0] + p_ref[1] + b_ref[...], 0.0)


def _relu_bias_packed(p, b):
    _, n8, d8 = p.shape
    blk = _pick_blk(n8)
    return pl.pallas_call(
        _relu_body,
        grid=(n8 // blk,),
        in_specs=[
            pl.BlockSpec((2, blk, d8), lambda i: (0, i, 0)),
            pl.BlockSpec((1, d8), lambda i: (0, 0)),
        ],
        out_specs=pl.BlockSpec((blk, d8), lambda i: (i, 0)),
        out_shape=jax.ShapeDtypeStruct((n8, d8), jnp.float32),
    )(p, b.reshape(1, d8))


def _final_body(p_ref, w_ref, b_ref, o_ref):
    z = jnp.dot(p_ref[0] + p_ref[1], w_ref[0], preferred_element_type=jnp.float32)
    z = z + b_ref[...]
    m = jnp.max(z, axis=1, keepdims=True)
    e = jnp.exp(z - m)
    s = jnp.sum(e, axis=1, keepdims=True)
    o_ref[0] = z - m - jnp.log(s)


def _final_packed(p, wbd, b):
    _, n8, d8 = p.shape
    _, _, m = wbd.shape
    blk = _pick_blk(n8)
    return pl.pallas_call(
        _final_body,
        grid=(n8 // blk, 8),
        in_specs=[
            pl.BlockSpec((2, blk, d8), lambda i, j: (0, i, 0)),
            pl.BlockSpec((1, d8, m), lambda i, j: (j, 0, 0)),
            pl.BlockSpec((1, m), lambda i, j: (0, 0)),
        ],
        out_specs=pl.BlockSpec((1, blk, m), lambda i, j: (j, i, 0)),
        out_shape=jax.ShapeDtypeStruct((8, n8, m), jnp.float32),
    )(p, wbd, b.reshape(1, m))


def kernel(edge_index, features, edge_weights, W0, b0, W1, b1, W2, b2):
    n_nodes, _ = features.shape
    n_edges = edge_index.shape[1]
    d = W0.shape[1]

    n_nodes_p = -(-n_nodes // (NS * 8)) * (NS * 8)

    per_w = -(-n_edges // (NW * CHUNK)) * CHUNK
    nchunks = per_w // CHUNK
    n_pad = NW * per_w
    pad = n_pad - n_edges

    row = jnp.concatenate([edge_index[0], jnp.zeros((pad,), jnp.int32)])
    col = jnp.concatenate([edge_index[1], jnp.zeros((pad,), jnp.int32)])
    wts = jnp.concatenate([edge_weights, jnp.zeros((pad,), jnp.float32)])
    ntot = n_pad // CHUNK
    row3 = row.reshape(ntot, CROWS, 128)
    col3 = col.reshape(ntot, CROWS, 128)
    wts3 = lax.bitcast_convert_type(wts, jnp.int32).reshape(ntot, CROWS, 128)
    zpad = jnp.zeros((ntot, PKROWS - 3 * CROWS, 128), jnp.int32)
    packed = jnp.concatenate([row3, col3, wts3, zpad], axis=1)
    zeros = jnp.zeros((n_nodes_p, d), jnp.float32)

    agg = _make_agg(n_nodes_p, d, nchunks)
    np8 = n_nodes_p // 8
    bd1 = jnp.kron(jnp.eye(8, dtype=jnp.float32), W1)
    bd2 = jnp.stack(
        [jnp.pad(W2, ((16 * j, 128 - 16 * j - d), (0, 0))) for j in range(8)]
    )

    xw0 = _matmul(features, W0)
    xw0 = jnp.pad(xw0, ((0, n_nodes_p - n_nodes), (0, 0)))
    p0 = agg(zeros, xw0, packed)
    pp0 = p0.reshape(NC, np8, 8 * d)
    xw1 = _relu_matmul_packed(pp0, jnp.tile(b0, 8), bd1)
    p1 = agg(zeros, xw1.reshape(n_nodes_p, d), packed)
    pp1 = p1.reshape(NC, np8, 8 * d)
    h1 = _relu_bias_packed(pp1, jnp.tile(b1, 8))
    p2 = agg(zeros, h1.reshape(n_nodes_p, d), packed)
    pp2 = p2.reshape(NC, np8, 8 * d)
    outp = _final_packed(pp2, bd2, b2)
    out = outp.transpose(1, 0, 2).reshape(n_nodes_p, 64)
    return out[:n_nodes]
